# baseline (speedup 1.0000x reference)
import functools

import jax
import jax.numpy as jnp
from jax.experimental import pallas as pl


def _gating_block_kernel(emb_ref, maskf_ref, w1_ref, b1_ref, w2_ref,
                         b2_ref, wc1_ref, bc1_ref, wc2_ref, bc2_ref,
                         prob_ref, weights_ref, *, n_experts):
    w1 = w1_ref[:]
    b1 = b1_ref[:]
    w2 = w2_ref[:]
    maskf = maskf_ref[:]

    logit_cols = []
    for e in range(n_experts):
        x16 = emb_ref[e].astype(jnp.bfloat16)
        h16 = jnp.maximum(
            jnp.dot(x16, w1, preferred_element_type=jnp.float32) + b1,
            0.0).astype(jnp.bfloat16)
        logit_cols.append(
            jnp.dot(h16, w2, preferred_element_type=jnp.float32))
    logits = jnp.concatenate(logit_cols, axis=1) + b2_ref[0, 0]

    neg = jnp.float32(-1e30)
    ml = jnp.where(maskf > 0.0, logits, neg)
    m = jnp.max(ml, axis=1, keepdims=True)
    ex = jnp.exp(ml - m) * maskf
    denom = jnp.sum(ex, axis=1, keepdims=True)
    any_valid = denom > 0.0
    weights = jnp.where(any_valid, ex / jnp.where(any_valid, denom, 1.0), 0.0)
    weights_ref[:] = weights

    fused = weights[:, 0:1] * emb_ref[0]
    for e in range(1, n_experts):
        fused = fused + weights[:, e:e + 1] * emb_ref[e]

    hc16 = jnp.maximum(
        jnp.dot(fused.astype(jnp.bfloat16), wc1_ref[:],
                preferred_element_type=jnp.float32) + bc1_ref[:],
        0.0).astype(jnp.bfloat16)
    z = jnp.dot(hc16, wc2_ref[:],
                preferred_element_type=jnp.float32) + bc2_ref[0, 0]
    prob_ref[:] = jax.nn.sigmoid(z)


@functools.partial(jax.jit, static_argnames=())
def kernel(expert_embeddings, mask, W1, b1, W2, b2, Wc1, bc1, Wc2, bc2):
    E, B, D = expert_embeddings.shape
    H = W1.shape[1]
    bm = min(256, B)
    assert B % bm == 0
    grid = (B // bm,)

    maskf = mask.astype(jnp.float32)
    W1h = W1.astype(jnp.bfloat16)
    Wc1h = Wc1.astype(jnp.bfloat16)
    W2h = W2.astype(jnp.bfloat16)
    Wc2h = Wc2.astype(jnp.bfloat16)
    b1r = b1.reshape(1, H)
    bc1r = bc1.reshape(1, H)
    b2r = b2.reshape(1, 1)
    bc2r = bc2.reshape(1, 1)

    out_shapes = (
        jax.ShapeDtypeStruct((B, 1), jnp.float32),
        jax.ShapeDtypeStruct((B, E), jnp.float32),
    )
    in_specs = [
        pl.BlockSpec((E, bm, D), lambda i: (0, i, 0)),
        pl.BlockSpec((bm, E), lambda i: (i, 0)),
        pl.BlockSpec((D, H), lambda i: (0, 0)),
        pl.BlockSpec((1, H), lambda i: (0, 0)),
        pl.BlockSpec((H, 1), lambda i: (0, 0)),
        pl.BlockSpec((1, 1), lambda i: (0, 0)),
        pl.BlockSpec((D, H), lambda i: (0, 0)),
        pl.BlockSpec((1, H), lambda i: (0, 0)),
        pl.BlockSpec((H, 1), lambda i: (0, 0)),
        pl.BlockSpec((1, 1), lambda i: (0, 0)),
    ]
    out_specs = (
        pl.BlockSpec((bm, 1), lambda i: (i, 0)),
        pl.BlockSpec((bm, E), lambda i: (i, 0)),
    )

    final_prob, weights = pl.pallas_call(
        functools.partial(_gating_block_kernel, n_experts=E),
        grid=grid,
        in_specs=in_specs,
        out_specs=out_specs,
        out_shape=out_shapes,
    )(expert_embeddings, maskf, W1h, b1r, W2h, b2r, Wc1h, bc1r, Wc2h, bc2r)

    return final_prob, weights


# speedup vs baseline: 1.0617x; 1.0617x over previous
import functools

import jax
import jax.numpy as jnp
from jax import lax
from jax.experimental import pallas as pl
from jax.experimental.pallas import tpu as pltpu
from jax.experimental.pallas import tpu_sc as plsc


def _scorer_kernel(emb_ref, w1_ref, b1_ref, w2_ref, b2_ref, logits_ref,
                   *, n_experts):
    w1 = w1_ref[:]
    b1 = b1_ref[:]
    w2 = w2_ref[:]
    rows = []
    for e in range(n_experts):
        h = jnp.maximum(jnp.dot(emb_ref[e], w1) + b1, 0.0)
        rows.append(jnp.sum(h * w2, axis=1))
    logits_ref[:] = jnp.stack(rows, axis=0) + b2_ref[0, 0]


def _fusion_kernel(emb_ref, wT_ref, wc1_ref, bc1_ref, wc2_ref, bc2_ref,
                   prob_ref, weights_ref, *, n_experts):
    weights = jnp.transpose(wT_ref[:])
    weights_ref[:] = weights
    fused = weights[:, 0:1] * emb_ref[0]
    for e in range(1, n_experts):
        fused = fused + weights[:, e:e + 1] * emb_ref[e]
    hc = jnp.maximum(jnp.dot(fused, wc1_ref[:]) + bc1_ref[:], 0.0)
    z = jnp.sum(hc * wc2_ref[:], axis=1, keepdims=True) + bc2_ref[0, 0]
    prob_ref[:] = jax.nn.sigmoid(z)


def _make_sc_softmax(n_experts, n_tokens):
    info = plsc.get_sparse_core_info()
    nworkers = info.num_cores * info.num_subcores
    lanes = info.num_lanes
    per_w = n_tokens // nworkers
    mesh = plsc.VectorSubcoreMesh(core_axis_name="c", subcore_axis_name="s")

    @functools.partial(
        pl.kernel, mesh=mesh,
        out_type=jax.ShapeDtypeStruct((n_experts, n_tokens), jnp.float32),
        scratch_types=[
            pltpu.VMEM((n_experts, per_w), jnp.float32),
            pltpu.VMEM((n_experts, per_w), jnp.float32),
            pltpu.VMEM((n_experts, per_w), jnp.float32),
        ],
    )
    def sc_softmax(logits_hbm, maskf_hbm, out_hbm, lg_v, mk_v, wt_v):
        wid = lax.axis_index("s") * info.num_cores + lax.axis_index("c")
        base = wid * per_w
        for e in range(n_experts):
            pltpu.sync_copy(logits_hbm.at[e, pl.ds(base, per_w)], lg_v.at[e])
            pltpu.sync_copy(maskf_hbm.at[e, pl.ds(base, per_w)], mk_v.at[e])
        neg = jnp.float32(-1e30)
        for c in range(per_w // lanes):
            sl = pl.ds(c * lanes, lanes)
            vs, ms = [], []
            for e in range(n_experts):
                ms.append(mk_v[e, sl])
                vs.append(jnp.where(ms[e] > 0.0, lg_v[e, sl], neg))
            m = vs[0]
            for e in range(1, n_experts):
                m = jnp.maximum(m, vs[e])
            exs = [jnp.exp(vs[e] - m) * ms[e] for e in range(n_experts)]
            denom = exs[0]
            for e in range(1, n_experts):
                denom = denom + exs[e]
            valid = denom > 0.0
            inv = jnp.where(valid, 1.0, 0.0) / jnp.where(valid, denom, 1.0)
            for e in range(n_experts):
                wt_v[e, sl] = exs[e] * inv
        for e in range(n_experts):
            pltpu.sync_copy(wt_v.at[e], out_hbm.at[e, pl.ds(base, per_w)])

    return sc_softmax


@functools.partial(jax.jit, static_argnames=())
def kernel(expert_embeddings, mask, W1, b1, W2, b2, Wc1, bc1, Wc2, bc2):
    E, B, D = expert_embeddings.shape
    H = W1.shape[1]
    bm = min(256, B)
    assert B % bm == 0
    grid = (B // bm,)

    maskfT = mask.astype(jnp.float32).T
    b1r = b1.reshape(1, H)
    w2r = W2.reshape(1, H)
    bc1r = bc1.reshape(1, H)
    wc2r = Wc2.reshape(1, H)
    b2r = b2.reshape(1, 1)
    bc2r = bc2.reshape(1, 1)

    logitsT = pl.pallas_call(
        functools.partial(_scorer_kernel, n_experts=E),
        grid=grid,
        in_specs=[
            pl.BlockSpec((E, bm, D), lambda i: (0, i, 0)),
            pl.BlockSpec((D, H), lambda i: (0, 0)),
            pl.BlockSpec((1, H), lambda i: (0, 0)),
            pl.BlockSpec((1, H), lambda i: (0, 0)),
            pl.BlockSpec((1, 1), lambda i: (0, 0)),
        ],
        out_specs=pl.BlockSpec((E, bm), lambda i: (0, i)),
        out_shape=jax.ShapeDtypeStruct((E, B), jnp.float32),
    )(expert_embeddings, W1, b1r, w2r, b2r)

    weightsT = _make_sc_softmax(E, B)(logitsT, maskfT)

    final_prob, weights = pl.pallas_call(
        functools.partial(_fusion_kernel, n_experts=E),
        grid=grid,
        in_specs=[
            pl.BlockSpec((E, bm, D), lambda i: (0, i, 0)),
            pl.BlockSpec((E, bm), lambda i: (0, i)),
            pl.BlockSpec((D, H), lambda i: (0, 0)),
            pl.BlockSpec((1, H), lambda i: (0, 0)),
            pl.BlockSpec((1, H), lambda i: (0, 0)),
            pl.BlockSpec((1, 1), lambda i: (0, 0)),
        ],
        out_specs=(
            pl.BlockSpec((bm, 1), lambda i: (i, 0)),
            pl.BlockSpec((bm, E), lambda i: (i, 0)),
        ),
        out_shape=(
            jax.ShapeDtypeStruct((B, 1), jnp.float32),
            jax.ShapeDtypeStruct((B, E), jnp.float32),
        ),
    )(expert_embeddings, weightsT, Wc1, bc1r, wc2r, bc2r)

    return final_prob, weights


# speedup vs baseline: 1.1496x; 1.0828x over previous
import functools

import jax
import jax.numpy as jnp
from jax import lax
from jax.experimental import pallas as pl
from jax.experimental.pallas import tpu as pltpu
from jax.experimental.pallas import tpu_sc as plsc


def _gating_block_kernel(emb_ref, maskf_ref, w1_ref, b1_ref, w2_ref,
                         b2_ref, wc1_ref, bc1_ref, wc2_ref, bc2_ref,
                         prob_ref, logitsT_ref, *, n_experts):
    w1 = w1_ref[:]
    b1 = b1_ref[:]
    w2 = w2_ref[:]
    maskf = maskf_ref[:]

    logits_cols = []
    for e in range(n_experts):
        x_e = emb_ref[e]
        h = jnp.maximum(jnp.dot(x_e, w1) + b1, 0.0)
        logits_cols.append(jnp.sum(h * w2, axis=1))
    logits = jnp.stack(logits_cols, axis=1) + b2_ref[0, 0]
    logitsT_ref[:] = jnp.transpose(logits)

    neg = jnp.float32(-1e30)
    ml = jnp.where(maskf > 0.0, logits, neg)
    m = jnp.max(ml, axis=1, keepdims=True)
    ex = jnp.exp(ml - m) * maskf
    denom = jnp.sum(ex, axis=1, keepdims=True)
    any_valid = denom > 0.0
    weights = jnp.where(any_valid, ex / jnp.where(any_valid, denom, 1.0), 0.0)

    fused = weights[:, 0:1] * emb_ref[0]
    for e in range(1, n_experts):
        fused = fused + weights[:, e:e + 1] * emb_ref[e]

    hc = jnp.maximum(jnp.dot(fused, wc1_ref[:]) + bc1_ref[:], 0.0)
    z = jnp.sum(hc * wc2_ref[:], axis=1, keepdims=True) + bc2_ref[0, 0]
    prob_ref[:] = jax.nn.sigmoid(z)


def _make_sc_softmax(n_experts, n_tokens):
    info = plsc.get_sparse_core_info()
    nworkers = info.num_cores * info.num_subcores
    lanes = info.num_lanes
    per_w = n_tokens // nworkers
    mesh = plsc.VectorSubcoreMesh(core_axis_name="c", subcore_axis_name="s")

    @functools.partial(
        pl.kernel, mesh=mesh,
        out_type=jax.ShapeDtypeStruct((n_experts, n_tokens), jnp.float32),
        scratch_types=[
            pltpu.VMEM((n_experts, per_w), jnp.float32),
            pltpu.VMEM((n_experts, per_w), jnp.float32),
            pltpu.VMEM((n_experts, per_w), jnp.float32),
        ],
    )
    def sc_softmax(logits_hbm, maskf_hbm, out_hbm, lg_v, mk_v, wt_v):
        wid = lax.axis_index("s") * info.num_cores + lax.axis_index("c")
        base = wid * per_w
        for e in range(n_experts):
            pltpu.sync_copy(logits_hbm.at[e, pl.ds(base, per_w)], lg_v.at[e])
            pltpu.sync_copy(maskf_hbm.at[e, pl.ds(base, per_w)], mk_v.at[e])
        neg = jnp.float32(-1e30)
        for c in range(per_w // lanes):
            sl = pl.ds(c * lanes, lanes)
            vs, ms = [], []
            for e in range(n_experts):
                ms.append(mk_v[e, sl])
                vs.append(jnp.where(ms[e] > 0.0, lg_v[e, sl], neg))
            m = vs[0]
            for e in range(1, n_experts):
                m = jnp.maximum(m, vs[e])
            exs = [jnp.exp(vs[e] - m) * ms[e] for e in range(n_experts)]
            denom = exs[0]
            for e in range(1, n_experts):
                denom = denom + exs[e]
            valid = denom > 0.0
            inv = jnp.where(valid, 1.0, 0.0) / jnp.where(valid, denom, 1.0)
            for e in range(n_experts):
                wt_v[e, sl] = exs[e] * inv
        for e in range(n_experts):
            pltpu.sync_copy(wt_v.at[e], out_hbm.at[e, pl.ds(base, per_w)])

    return sc_softmax


@functools.partial(jax.jit, static_argnames=())
def kernel(expert_embeddings, mask, W1, b1, W2, b2, Wc1, bc1, Wc2, bc2):
    E, B, D = expert_embeddings.shape
    H = W1.shape[1]
    bm = min(256, B)
    assert B % bm == 0
    grid = (B // bm,)

    maskf = mask.astype(jnp.float32)
    maskfT = maskf.T
    b1r = b1.reshape(1, H)
    w2r = W2.reshape(1, H)
    bc1r = bc1.reshape(1, H)
    wc2r = Wc2.reshape(1, H)
    b2r = b2.reshape(1, 1)
    bc2r = bc2.reshape(1, 1)

    out_shapes = (
        jax.ShapeDtypeStruct((B, 1), jnp.float32),
        jax.ShapeDtypeStruct((E, B), jnp.float32),
    )
    in_specs = [
        pl.BlockSpec((E, bm, D), lambda i: (0, i, 0)),
        pl.BlockSpec((bm, E), lambda i: (i, 0)),
        pl.BlockSpec((D, H), lambda i: (0, 0)),
        pl.BlockSpec((1, H), lambda i: (0, 0)),
        pl.BlockSpec((1, H), lambda i: (0, 0)),
        pl.BlockSpec((1, 1), lambda i: (0, 0)),
        pl.BlockSpec((D, H), lambda i: (0, 0)),
        pl.BlockSpec((1, H), lambda i: (0, 0)),
        pl.BlockSpec((1, H), lambda i: (0, 0)),
        pl.BlockSpec((1, 1), lambda i: (0, 0)),
    ]
    out_specs = (
        pl.BlockSpec((bm, 1), lambda i: (i, 0)),
        pl.BlockSpec((E, bm), lambda i: (0, i)),
    )

    final_prob, logitsT = pl.pallas_call(
        functools.partial(_gating_block_kernel, n_experts=E),
        grid=grid,
        in_specs=in_specs,
        out_specs=out_specs,
        out_shape=out_shapes,
    )(expert_embeddings, maskf, W1, b1r, w2r, b2r, Wc1, bc1r, wc2r, bc2r)

    weightsT = _make_sc_softmax(E, B)(logitsT, maskfT)
    weights = weightsT.T

    return final_prob, weights
